# baseline (speedup 1.0000x reference)
import functools

import jax
import jax.numpy as jnp
from jax import lax
from jax.experimental import pallas as pl
from jax.experimental.pallas import tpu as pltpu
from jax.experimental.pallas import tpu_sc as plsc

B = 16384
EMB = 32
HID = 32
PK = 4
W = PK * EMB

_info = plsc.get_sparse_core_info()
_NC, _NS = _info.num_cores, _info.num_subcores
_NW = _NC * _NS
_BPW = B // _NW
_NR = 2
_RPW = _BPW // _NR

_mesh = plsc.VectorSubcoreMesh(core_axis_name="c", subcore_axis_name="s")


_TCOL_BITS = 13
_TCOL = 1 << _TCOL_BITS
_TROW_BITS = _TCOL_BITS - 2
_TROW = _TCOL // PK


def _transpose_body(xt_ref, out_ref):
    x = xt_ref[...]
    r = lax.broadcasted_iota(jnp.int32, (EMB, EMB), 0)
    c = lax.broadcasted_iota(jnp.int32, (EMB, EMB), 1)
    eye = (r == c).astype(jnp.float32)
    for j in range(PK):
        xs = x[:, j * _TROW:(j + 1) * _TROW]
        y = lax.dot_general(xs, eye, (((0,), (0,)), ((), ())),
                            preferred_element_type=jnp.float32)
        out_ref[:, j * EMB:(j + 1) * EMB] = y.astype(jnp.bfloat16)


def _pack_rows(tab_t, n):
    grid = (pl.cdiv(n, _TCOL),)
    return pl.pallas_call(
        _transpose_body,
        grid=grid,
        in_specs=[pl.BlockSpec((EMB, _TCOL), lambda i: (0, i))],
        out_specs=pl.BlockSpec((_TROW, W), lambda i: (i, 0)),
        out_shape=jax.ShapeDtypeStruct((pl.cdiv(n, _TCOL) * _TROW, W),
                                       jnp.bfloat16),
    )(tab_t)


def _packed_row(idx):
    return (lax.shift_right_logical(idx, _TCOL_BITS) * _TROW) + \
        (idx & (_TROW - 1))


def _packed_sub(idx):
    return lax.shift_right_logical(idx, _TROW_BITS) & (PK - 1)


@functools.partial(
    pl.kernel,
    mesh=_mesh,
    compiler_params=pltpu.CompilerParams(use_tc_tiling_on_sc=False),
    out_type=[
        jax.ShapeDtypeStruct((B, W), jnp.bfloat16),
        jax.ShapeDtypeStruct((B, W), jnp.bfloat16),
    ],
    scratch_types=[
        pltpu.VMEM((_BPW,), jnp.int32),
        pltpu.VMEM((_BPW,), jnp.int32),
        pltpu.VMEM((_RPW, W), jnp.bfloat16),
        pltpu.VMEM((_RPW, W), jnp.bfloat16),
        pltpu.SemaphoreType.DMA,
        pltpu.SemaphoreType.DMA,
    ],
)
def _sc_gather(u_tid_hbm, i_tid_hbm, u_tab_hbm, i_tab_hbm,
               out_u_hbm, out_i_hbm,
               uidx_v, iidx_v, urows_v, irows_v, sem_u, sem_i):
    wid = lax.axis_index("s") * _NC + lax.axis_index("c")
    base = wid * _BPW
    pltpu.sync_copy(u_tid_hbm.at[pl.ds(base, _BPW)], uidx_v)
    pltpu.sync_copy(i_tid_hbm.at[pl.ds(base, _BPW)], iidx_v)
    for r in range(_NR):
        cu = pltpu.async_copy(u_tab_hbm.at[uidx_v.at[pl.ds(r * _RPW, _RPW)]],
                              urows_v, sem_u)
        ci = pltpu.async_copy(i_tab_hbm.at[iidx_v.at[pl.ds(r * _RPW, _RPW)]],
                              irows_v, sem_i)
        cu.wait()
        ci.wait()
        pltpu.sync_copy(urows_v, out_u_hbm.at[pl.ds(base + r * _RPW, _RPW)])
        pltpu.sync_copy(irows_v, out_i_hbm.at[pl.ds(base + r * _RPW, _RPW)])


_BLK = 2048


def _pick(rows, m):
    sel = jnp.where(m == 0, rows[:, 0:EMB], rows[:, EMB:2 * EMB])
    sel = jnp.where(m == 2, rows[:, 2 * EMB:3 * EMB], sel)
    return jnp.where(m == 3, rows[:, 3 * EMB:4 * EMB], sel)


def _mlp_body(ur_ref, ir_ref, uid_ref, iid_ref, uat_ref, iat_ref,
              w1_ref, b1_ref, w2_ref, b2_ref, out_ref):
    uf = _pick(ur_ref[...], _packed_sub(uid_ref[...])).astype(jnp.float32)
    itf = _pick(ir_ref[...], _packed_sub(iid_ref[...])).astype(jnp.float32)
    w1 = w1_ref[...]
    acc = jnp.dot(uf, w1[0:EMB, :], preferred_element_type=jnp.float32)
    acc += jnp.dot(itf, w1[EMB:2 * EMB, :], preferred_element_type=jnp.float32)
    acc += jnp.dot(jnp.transpose(uat_ref[...], (1, 0)), w1[2 * EMB:3 * EMB, :],
                   preferred_element_type=jnp.float32)
    acc += jnp.dot(jnp.transpose(iat_ref[...], (1, 0)), w1[3 * EMB:4 * EMB, :],
                   preferred_element_type=jnp.float32)
    h = jnp.maximum(acc + b1_ref[...], 0.0)
    o = jnp.dot(h, w2_ref[...], preferred_element_type=jnp.float32)
    out_ref[...] = o + b2_ref[0, 0]


def _mlp(ur, ir, uid, iid, uat, iat, w1, b1, w2, b2):
    grid = (B // _BLK,)
    return pl.pallas_call(
        _mlp_body,
        grid=grid,
        in_specs=[
            pl.BlockSpec((_BLK, W), lambda i: (i, 0)),
            pl.BlockSpec((_BLK, W), lambda i: (i, 0)),
            pl.BlockSpec((_BLK, 1), lambda i: (i, 0)),
            pl.BlockSpec((_BLK, 1), lambda i: (i, 0)),
            pl.BlockSpec((EMB, _BLK), lambda i: (0, i)),
            pl.BlockSpec((EMB, _BLK), lambda i: (0, i)),
            pl.BlockSpec((4 * EMB, HID), lambda i: (0, 0)),
            pl.BlockSpec((1, HID), lambda i: (0, 0)),
            pl.BlockSpec((HID, 1), lambda i: (0, 0)),
            pl.BlockSpec((1, 1), lambda i: (0, 0)),
        ],
        out_specs=pl.BlockSpec((_BLK, 1), lambda i: (i, 0)),
        out_shape=jax.ShapeDtypeStruct((B, 1), jnp.float32),
    )(ur, ir, uid, iid, uat, iat, w1, b1, w2, b2)


def kernel(u_id, i_id, u_attr, i_attr, user_table, item_table, W1, b1, W2, b2):
    uid = u_id.astype(jnp.int32)
    iid = i_id.astype(jnp.int32)
    ut4 = _pack_rows(user_table.T, user_table.shape[0])
    it4 = _pack_rows(item_table.T, item_table.shape[0])
    ur, ir = _sc_gather(_packed_row(uid), _packed_row(iid), ut4, it4)
    uat = jnp.squeeze(u_attr, axis=1).T
    iat = jnp.squeeze(i_attr, axis=1).T
    out = _mlp(ur, ir, uid.reshape(B, 1), iid.reshape(B, 1), uat, iat,
               W1, b1.reshape(1, HID), W2, b2.reshape(1, 1))
    return jnp.squeeze(out, axis=1)


# speedup vs baseline: 1.7604x; 1.7604x over previous
import functools

import jax
import jax.numpy as jnp
from jax import lax
from jax.experimental import pallas as pl
from jax.experimental.pallas import tpu as pltpu
from jax.experimental.pallas import tpu_sc as plsc

B = 16384
EMB = 32
HID = 32
PK = 4
W = PK * EMB

_info = plsc.get_sparse_core_info()
_NC, _NS = _info.num_cores, _info.num_subcores
_NW = _NC * _NS
_BPW = B // _NW
_NR = 2
_RPW = _BPW // _NR

_mesh = plsc.VectorSubcoreMesh(core_axis_name="c", subcore_axis_name="s")


_TCOL_BITS = 15
_TCOL = 1 << _TCOL_BITS
_TROW_BITS = _TCOL_BITS - 2
_TROW = _TCOL // PK


def _transpose_body(xt_ref, out_ref):
    x = xt_ref[...]
    r = lax.broadcasted_iota(jnp.int32, (EMB, EMB), 0)
    c = lax.broadcasted_iota(jnp.int32, (EMB, EMB), 1)
    eye = (r == c).astype(jnp.float32)
    del eye
    y = jnp.transpose(x, (1, 0))
    out_ref[...] = jnp.concatenate(
        [y[j * _TROW:(j + 1) * _TROW, :] for j in range(PK)], axis=1)


def _pack_rows(tab_t, n):
    grid = (pl.cdiv(n, _TCOL),)
    return pl.pallas_call(
        _transpose_body,
        grid=grid,
        in_specs=[pl.BlockSpec((EMB, _TCOL), lambda i: (0, i))],
        out_specs=pl.BlockSpec((_TROW, W), lambda i: (i, 0)),
        out_shape=jax.ShapeDtypeStruct((pl.cdiv(n, _TCOL) * _TROW, W),
                                       jnp.float32),
    )(tab_t)


def _packed_row(idx):
    return (lax.shift_right_logical(idx, _TCOL_BITS) * _TROW) + \
        (idx & (_TROW - 1))


def _packed_sub(idx):
    return lax.shift_right_logical(idx, _TROW_BITS) & (PK - 1)


@functools.partial(
    pl.kernel,
    mesh=_mesh,
    compiler_params=pltpu.CompilerParams(use_tc_tiling_on_sc=False),
    out_type=jax.ShapeDtypeStruct((B, W), jnp.float32),
    scratch_types=[
        pltpu.VMEM((_BPW,), jnp.int32),
        pltpu.VMEM((_RPW, W), jnp.float32),
        pltpu.VMEM((_RPW, W), jnp.float32),
        pltpu.SemaphoreType.DMA,
        pltpu.SemaphoreType.DMA,
    ],
)
def _sc_gather(tid_hbm, tab_hbm, out_hbm, idx_v, rows_a, rows_b, sem_a, sem_b):
    wid = lax.axis_index("s") * _NC + lax.axis_index("c")
    base = wid * _BPW
    pltpu.sync_copy(tid_hbm.at[pl.ds(base, _BPW)], idx_v)
    ca = pltpu.async_copy(tab_hbm.at[idx_v.at[pl.ds(0, _RPW)]], rows_a, sem_a)
    cb = pltpu.async_copy(tab_hbm.at[idx_v.at[pl.ds(_RPW, _RPW)]], rows_b,
                          sem_b)
    ca.wait()
    pltpu.sync_copy(rows_a, out_hbm.at[pl.ds(base, _RPW)])
    cb.wait()
    pltpu.sync_copy(rows_b, out_hbm.at[pl.ds(base + _RPW, _RPW)])


_BLK = 2048


def _pick(rows, m):
    sel = jnp.where(m == 0, rows[:, 0:EMB], rows[:, EMB:2 * EMB])
    sel = jnp.where(m == 2, rows[:, 2 * EMB:3 * EMB], sel)
    return jnp.where(m == 3, rows[:, 3 * EMB:4 * EMB], sel)


def _mlp_body(ur_ref, ir_ref, uid_ref, iid_ref, uat_ref, iat_ref,
              w1_ref, b1_ref, w2_ref, b2_ref, out_ref):
    uf = _pick(ur_ref[...], _packed_sub(uid_ref[...]))
    itf = _pick(ir_ref[...], _packed_sub(iid_ref[...]))
    w1 = w1_ref[...]
    acc = jnp.dot(uf, w1[0:EMB, :], preferred_element_type=jnp.float32)
    acc += jnp.dot(itf, w1[EMB:2 * EMB, :], preferred_element_type=jnp.float32)
    acc += jnp.dot(jnp.transpose(uat_ref[...], (1, 0)), w1[2 * EMB:3 * EMB, :],
                   preferred_element_type=jnp.float32)
    acc += jnp.dot(jnp.transpose(iat_ref[...], (1, 0)), w1[3 * EMB:4 * EMB, :],
                   preferred_element_type=jnp.float32)
    h = jnp.maximum(acc + b1_ref[...], 0.0)
    o = jnp.dot(h, w2_ref[...], preferred_element_type=jnp.float32)
    out_ref[...] = o + b2_ref[0, 0]


def _mlp(ur, ir, uid, iid, uat, iat, w1, b1, w2, b2):
    grid = (B // _BLK,)
    return pl.pallas_call(
        _mlp_body,
        grid=grid,
        in_specs=[
            pl.BlockSpec((_BLK, W), lambda i: (i, 0)),
            pl.BlockSpec((_BLK, W), lambda i: (i, 0)),
            pl.BlockSpec((_BLK, 1), lambda i: (i, 0)),
            pl.BlockSpec((_BLK, 1), lambda i: (i, 0)),
            pl.BlockSpec((EMB, _BLK), lambda i: (0, i)),
            pl.BlockSpec((EMB, _BLK), lambda i: (0, i)),
            pl.BlockSpec((4 * EMB, HID), lambda i: (0, 0)),
            pl.BlockSpec((1, HID), lambda i: (0, 0)),
            pl.BlockSpec((HID, 1), lambda i: (0, 0)),
            pl.BlockSpec((1, 1), lambda i: (0, 0)),
        ],
        out_specs=pl.BlockSpec((_BLK, 1), lambda i: (i, 0)),
        out_shape=jax.ShapeDtypeStruct((B, 1), jnp.float32),
    )(ur, ir, uid, iid, uat, iat, w1, b1, w2, b2)


def kernel(u_id, i_id, u_attr, i_attr, user_table, item_table, W1, b1, W2, b2):
    uid = u_id.astype(jnp.int32)
    iid = i_id.astype(jnp.int32)
    ut4 = _pack_rows(user_table.T, user_table.shape[0])
    it4 = _pack_rows(item_table.T, item_table.shape[0])
    ir = _sc_gather(_packed_row(iid), it4)
    ur = _sc_gather(_packed_row(uid), ut4)
    uat = jnp.squeeze(u_attr, axis=1).T
    iat = jnp.squeeze(i_attr, axis=1).T
    out = _mlp(ur, ir, uid.reshape(B, 1), iid.reshape(B, 1), uat, iat,
               W1, b1.reshape(1, HID), W2, b2.reshape(1, 1))
    return jnp.squeeze(out, axis=1)


# speedup vs baseline: 1.7643x; 1.0023x over previous
import functools

import jax
import jax.numpy as jnp
from jax import lax
from jax.experimental import pallas as pl
from jax.experimental.pallas import tpu as pltpu
from jax.experimental.pallas import tpu_sc as plsc

B = 16384
EMB = 32
HID = 32
PK = 4
W = PK * EMB

_info = plsc.get_sparse_core_info()
_NC, _NS = _info.num_cores, _info.num_subcores
_NW = _NC * _NS
_BPW = B // _NW
_NR = 2
_RPW = _BPW // _NR

_mesh = plsc.VectorSubcoreMesh(core_axis_name="c", subcore_axis_name="s")


_TCOL_BITS = 13
_TCOL = 1 << _TCOL_BITS
_TROW_BITS = _TCOL_BITS - 2
_TROW = _TCOL // PK


def _transpose_body(xt_ref, out_ref):
    x = xt_ref[...]
    r = lax.broadcasted_iota(jnp.int32, (EMB, EMB), 0)
    c = lax.broadcasted_iota(jnp.int32, (EMB, EMB), 1)
    eye = (r == c).astype(jnp.float32)
    del eye
    y = jnp.transpose(x, (1, 0))
    out_ref[...] = jnp.concatenate(
        [y[j * _TROW:(j + 1) * _TROW, :] for j in range(PK)], axis=1)


def _pack_rows(tab_t, n):
    grid = (pl.cdiv(n, _TCOL),)
    return pl.pallas_call(
        _transpose_body,
        grid=grid,
        in_specs=[pl.BlockSpec((EMB, _TCOL), lambda i: (0, i))],
        out_specs=pl.BlockSpec((_TROW, W), lambda i: (i, 0)),
        out_shape=jax.ShapeDtypeStruct((pl.cdiv(n, _TCOL) * _TROW, W),
                                       jnp.float32),
    )(tab_t)


def _packed_row(idx):
    return (lax.shift_right_logical(idx, _TCOL_BITS) * _TROW) + \
        (idx & (_TROW - 1))


def _packed_sub(idx):
    return lax.shift_right_logical(idx, _TROW_BITS) & (PK - 1)


@functools.partial(
    pl.kernel,
    mesh=_mesh,
    compiler_params=pltpu.CompilerParams(use_tc_tiling_on_sc=False),
    out_type=jax.ShapeDtypeStruct((B, W), jnp.float32),
    scratch_types=[
        pltpu.VMEM((_BPW,), jnp.int32),
        pltpu.VMEM((_RPW, W), jnp.float32),
        pltpu.VMEM((_RPW, W), jnp.float32),
        pltpu.SemaphoreType.DMA,
        pltpu.SemaphoreType.DMA,
    ],
)
def _sc_gather(tid_hbm, tab_hbm, out_hbm, idx_v, rows_a, rows_b, sem_a, sem_b):
    wid = lax.axis_index("s") * _NC + lax.axis_index("c")
    base = wid * _BPW
    pltpu.sync_copy(tid_hbm.at[pl.ds(base, _BPW)], idx_v)
    ca = pltpu.async_copy(tab_hbm.at[idx_v.at[pl.ds(0, _RPW)]], rows_a, sem_a)
    cb = pltpu.async_copy(tab_hbm.at[idx_v.at[pl.ds(_RPW, _RPW)]], rows_b,
                          sem_b)
    ca.wait()
    pltpu.sync_copy(rows_a, out_hbm.at[pl.ds(base, _RPW)])
    cb.wait()
    pltpu.sync_copy(rows_b, out_hbm.at[pl.ds(base + _RPW, _RPW)])


_BLK = 2048


def _pick(rows, m):
    sel = jnp.where(m == 0, rows[:, 0:EMB], rows[:, EMB:2 * EMB])
    sel = jnp.where(m == 2, rows[:, 2 * EMB:3 * EMB], sel)
    return jnp.where(m == 3, rows[:, 3 * EMB:4 * EMB], sel)


def _mlp_body(ur_ref, ir_ref, uid_ref, iid_ref, uat_ref, iat_ref,
              w1_ref, b1_ref, w2_ref, b2_ref, out_ref):
    uf = _pick(ur_ref[...], _packed_sub(uid_ref[...]))
    itf = _pick(ir_ref[...], _packed_sub(iid_ref[...]))
    w1 = w1_ref[...]
    acc = jnp.dot(uf, w1[0:EMB, :], preferred_element_type=jnp.float32)
    acc += jnp.dot(itf, w1[EMB:2 * EMB, :], preferred_element_type=jnp.float32)
    acc += jnp.dot(jnp.transpose(uat_ref[...], (1, 0)), w1[2 * EMB:3 * EMB, :],
                   preferred_element_type=jnp.float32)
    acc += jnp.dot(jnp.transpose(iat_ref[...], (1, 0)), w1[3 * EMB:4 * EMB, :],
                   preferred_element_type=jnp.float32)
    h = jnp.maximum(acc + b1_ref[...], 0.0)
    o = jnp.dot(h, w2_ref[...], preferred_element_type=jnp.float32)
    out_ref[...] = o + b2_ref[0, 0]


def _mlp(ur, ir, uid, iid, uat, iat, w1, b1, w2, b2):
    grid = (B // _BLK,)
    return pl.pallas_call(
        _mlp_body,
        grid=grid,
        in_specs=[
            pl.BlockSpec((_BLK, W), lambda i: (i, 0)),
            pl.BlockSpec((_BLK, W), lambda i: (i, 0)),
            pl.BlockSpec((_BLK, 1), lambda i: (i, 0)),
            pl.BlockSpec((_BLK, 1), lambda i: (i, 0)),
            pl.BlockSpec((EMB, _BLK), lambda i: (0, i)),
            pl.BlockSpec((EMB, _BLK), lambda i: (0, i)),
            pl.BlockSpec((4 * EMB, HID), lambda i: (0, 0)),
            pl.BlockSpec((1, HID), lambda i: (0, 0)),
            pl.BlockSpec((HID, 1), lambda i: (0, 0)),
            pl.BlockSpec((1, 1), lambda i: (0, 0)),
        ],
        out_specs=pl.BlockSpec((_BLK, 1), lambda i: (i, 0)),
        out_shape=jax.ShapeDtypeStruct((B, 1), jnp.float32),
    )(ur, ir, uid, iid, uat, iat, w1, b1, w2, b2)


def kernel(u_id, i_id, u_attr, i_attr, user_table, item_table, W1, b1, W2, b2):
    uid = u_id.astype(jnp.int32)
    iid = i_id.astype(jnp.int32)
    it4 = _pack_rows(item_table.T, item_table.shape[0])
    ir = _sc_gather(_packed_row(iid), it4)
    ut4 = _pack_rows(user_table.T, user_table.shape[0])
    ur = _sc_gather(_packed_row(uid), ut4)
    uat = jnp.squeeze(u_attr, axis=1).T
    iat = jnp.squeeze(i_attr, axis=1).T
    out = _mlp(ur, ir, uid.reshape(B, 1), iid.reshape(B, 1), uat, iat,
               W1, b1.reshape(1, HID), W2, b2.reshape(1, 1))
    return jnp.squeeze(out, axis=1)
